# baseline (speedup 1.0000x reference)
import functools

import jax
import jax.numpy as jnp
from jax import lax
from jax.experimental import pallas as pl
from jax.experimental.pallas import tpu as pltpu
from jax.experimental.pallas import tpu_sc as plsc

NC = 2
NS = 16
NW = NC * NS
CHUNK = 128


DEPTH = 3
GROUP = 8


def _spmm_partials(x, src2d, dst2d, zeros):
    d = x.shape[1]
    npad = zeros.shape[0]
    r = src2d.shape[0]
    rows_per_sub = npad // NS
    rows_per_w = r // NW
    groups_per_w = rows_per_w // GROUP
    assert rows_per_w * NW == r and groups_per_w * GROUP == rows_per_w
    mesh = plsc.VectorSubcoreMesh(core_axis_name="c", subcore_axis_name="s")

    @functools.partial(
        pl.kernel,
        out_type=jax.ShapeDtypeStruct((NC, npad, d), jnp.float32),
        mesh=mesh,
        scratch_types=[
            [pltpu.VMEM((CHUNK,), jnp.int32) for _ in range(DEPTH)],
            [pltpu.VMEM((CHUNK,), jnp.int32) for _ in range(DEPTH)],
            [pltpu.VMEM((CHUNK, d), jnp.float32) for _ in range(DEPTH)],
            pltpu.VMEM_SHARED((npad, d), jnp.float32),
            [pltpu.SemaphoreType.DMA for _ in range(DEPTH)],
            [pltpu.SemaphoreType.DMA for _ in range(DEPTH)],
        ],
    )
    def spmm(x_hbm, src_hbm, dst_hbm, zero_hbm, out_hbm,
             sidx, didx, bufs, yacc, isems, gsems):
        c = lax.axis_index("c")
        s = lax.axis_index("s")
        wid = s * NC + c
        row0 = wid * rows_per_w
        pltpu.sync_copy(zero_hbm.at[pl.ds(s * rows_per_sub, rows_per_sub)],
                        yacc.at[pl.ds(s * rows_per_sub, rows_per_sub)])
        plsc.subcore_barrier()

        def group_body(g, carry):
            base = row0 + g * GROUP
            idm, gdm = {}, {}

            def istart(j, b):
                idm[j] = (
                    pltpu.async_copy(src_hbm.at[base + j], sidx[b], isems[b]),
                    pltpu.async_copy(dst_hbm.at[base + j], didx[b], isems[b]),
                )

            def gstart(j, b):
                idm[j][0].wait()
                gdm[j] = pltpu.async_copy(x_hbm.at[sidx[b]], bufs[b],
                                          gsems[b])

            def sdo(j, b):
                gdm[j].wait()
                idm[j][1].wait()
                pltpu.sync_copy(bufs[b], yacc.at[didx[b]], add=True)

            for p in range(DEPTH - 1):
                istart(p, p)
            gstart(0, 0)
            for j in range(GROUP):
                b = j % DEPTH
                nj = j + DEPTH - 1
                if nj < GROUP:
                    istart(nj, nj % DEPTH)
                if j + 1 < GROUP:
                    gstart(j + 1, (j + 1) % DEPTH)
                sdo(j, b)
            return carry

        lax.fori_loop(0, groups_per_w, group_body, 0)
        plsc.subcore_barrier()
        pltpu.sync_copy(yacc.at[pl.ds(s * rows_per_sub, rows_per_sub)],
                        out_hbm.at[c, pl.ds(s * rows_per_sub, rows_per_sub)])

    return spmm(x, src2d, dst2d, zeros)


def _mlp_body(y_ref, x_ref, w1_ref, b1_ref, w2_ref, b2_ref, scale_ref,
              g_ref, bt_ref, o_ref):
    n = x_ref.shape[0]
    h = y_ref[0, :n] + y_ref[1, :n] + scale_ref[...] * x_ref[...]
    h = jnp.dot(h, w1_ref[...], preferred_element_type=jnp.float32)
    h = jnp.maximum(h + b1_ref[...], 0.0)
    h = jnp.dot(h, w2_ref[...], preferred_element_type=jnp.float32)
    h = h + b2_ref[...]
    mean = jnp.mean(h, axis=0, keepdims=True)
    var = jnp.mean(jnp.square(h - mean), axis=0, keepdims=True)
    h = (h - mean) * lax.rsqrt(var + 1e-5) * g_ref[...] + bt_ref[...]
    o_ref[...] = jnp.maximum(h, 0.0)


def kernel(x, edge_index, W1, b1, W2, b2, eps, gamma, beta):
    n, d = x.shape
    e = edge_index.shape[1]
    assert e % CHUNK == 0
    npad = ((n + 8 * NS - 1) // (8 * NS)) * (8 * NS)
    r = e // CHUNK
    tile = GROUP * NW
    rpad = ((r + tile - 1) // tile) * tile
    dst2d = edge_index[0].reshape(r, CHUNK)
    src2d = edge_index[1].reshape(r, CHUNK)
    if rpad != r:
        dst2d = jnp.pad(dst2d, ((0, rpad - r), (0, 0)),
                        constant_values=npad - 1)
        src2d = jnp.pad(src2d, ((0, rpad - r), (0, 0)))
    zeros = jnp.zeros((npad, d), jnp.float32)

    partials = _spmm_partials(x, src2d, dst2d, zeros)

    scale = (1.0 + eps).reshape(1, 1)
    out = pl.pallas_call(
        _mlp_body,
        out_shape=jax.ShapeDtypeStruct((n, d), jnp.float32),
    )(partials, x, W1.T, b1.reshape(1, d), W2.T, b2.reshape(1, d),
      scale, gamma.reshape(1, d), beta.reshape(1, d))
    return out


# speedup vs baseline: 1.0008x; 1.0008x over previous
import functools

import jax
import jax.numpy as jnp
from jax import lax
from jax.experimental import pallas as pl
from jax.experimental.pallas import tpu as pltpu
from jax.experimental.pallas import tpu_sc as plsc

NC = 2
NS = 16
NW = NC * NS
CHUNK = 128


DEPTH = 3
GROUP = 8


def _spmm_partials(x, src2d, dst2d, zeros):
    d = x.shape[1]
    npad = zeros.shape[0]
    r = src2d.shape[0]
    rows_per_sub = npad // NS
    rows_per_w = r // NW
    groups_per_w = rows_per_w // GROUP
    assert rows_per_w * NW == r and groups_per_w * GROUP == rows_per_w
    mesh = plsc.VectorSubcoreMesh(core_axis_name="c", subcore_axis_name="s")

    @functools.partial(
        pl.kernel,
        out_type=jax.ShapeDtypeStruct((NC, npad, d), jnp.float32),
        mesh=mesh,
        scratch_types=[
            [pltpu.VMEM((CHUNK,), jnp.int32) for _ in range(DEPTH)],
            [pltpu.VMEM((CHUNK,), jnp.int32) for _ in range(DEPTH)],
            [pltpu.VMEM((CHUNK, d), jnp.float32) for _ in range(DEPTH)],
            pltpu.VMEM_SHARED((npad, d), jnp.float32),
            [pltpu.SemaphoreType.DMA for _ in range(DEPTH)],
            [pltpu.SemaphoreType.DMA for _ in range(DEPTH)],
        ],
    )
    def spmm(x_hbm, src_hbm, dst_hbm, zero_hbm, out_hbm,
             sidx, didx, bufs, yacc, isems, gsems):
        c = lax.axis_index("c")
        s = lax.axis_index("s")
        wid = s * NC + c
        row0 = wid * rows_per_w
        pltpu.sync_copy(zero_hbm.at[pl.ds(s * rows_per_sub, rows_per_sub)],
                        yacc.at[pl.ds(s * rows_per_sub, rows_per_sub)])
        plsc.subcore_barrier()

        def group_body(g, carry):
            base = row0 + g * GROUP
            idm, gdm = {}, {}

            def istart(j, b):
                idm[j] = (
                    pltpu.async_copy(src_hbm.at[base + j], sidx[b], isems[b]),
                    pltpu.async_copy(dst_hbm.at[base + j], didx[b], isems[b]),
                )

            def gstart(j, b):
                idm[j][0].wait()
                idm[j][1].wait()
                gdm[j] = pltpu.async_copy(x_hbm.at[sidx[b]], bufs[b],
                                          gsems[b])

            def sdo(j, b):
                gdm[j].wait()
                pltpu.sync_copy(bufs[b], yacc.at[didx[b]], add=True)

            for p in range(DEPTH - 1):
                istart(p, p)
            gstart(0, 0)
            for j in range(GROUP):
                b = j % DEPTH
                nj = j + DEPTH - 1
                if nj < GROUP:
                    istart(nj, nj % DEPTH)
                if j + 1 < GROUP:
                    gstart(j + 1, (j + 1) % DEPTH)
                sdo(j, b)
            return carry

        lax.fori_loop(0, groups_per_w, group_body, 0)
        plsc.subcore_barrier()
        pltpu.sync_copy(yacc.at[pl.ds(s * rows_per_sub, rows_per_sub)],
                        out_hbm.at[c, pl.ds(s * rows_per_sub, rows_per_sub)])

    return spmm(x, src2d, dst2d, zeros)


def _mlp_body(y_ref, x_ref, w1_ref, b1_ref, w2_ref, b2_ref, scale_ref,
              g_ref, bt_ref, o_ref):
    n = x_ref.shape[0]
    h = y_ref[0, :n] + y_ref[1, :n] + scale_ref[...] * x_ref[...]
    h = jnp.dot(h, w1_ref[...], preferred_element_type=jnp.float32)
    h = jnp.maximum(h + b1_ref[...], 0.0)
    h = jnp.dot(h, w2_ref[...], preferred_element_type=jnp.float32)
    h = h + b2_ref[...]
    mean = jnp.mean(h, axis=0, keepdims=True)
    var = jnp.mean(jnp.square(h - mean), axis=0, keepdims=True)
    h = (h - mean) * lax.rsqrt(var + 1e-5) * g_ref[...] + bt_ref[...]
    o_ref[...] = jnp.maximum(h, 0.0)


def kernel(x, edge_index, W1, b1, W2, b2, eps, gamma, beta):
    n, d = x.shape
    e = edge_index.shape[1]
    assert e % CHUNK == 0
    npad = ((n + 8 * NS - 1) // (8 * NS)) * (8 * NS)
    r = e // CHUNK
    tile = GROUP * NW
    rpad = ((r + tile - 1) // tile) * tile
    dst2d = edge_index[0].reshape(r, CHUNK)
    src2d = edge_index[1].reshape(r, CHUNK)
    if rpad != r:
        dst2d = jnp.pad(dst2d, ((0, rpad - r), (0, 0)),
                        constant_values=npad - 1)
        src2d = jnp.pad(src2d, ((0, rpad - r), (0, 0)))
    zeros = jnp.zeros((npad, d), jnp.float32)

    partials = _spmm_partials(x, src2d, dst2d, zeros)

    scale = (1.0 + eps).reshape(1, 1)
    out = pl.pallas_call(
        _mlp_body,
        out_shape=jax.ShapeDtypeStruct((n, d), jnp.float32),
    )(partials, x, W1.T, b1.reshape(1, d), W2.T, b2.reshape(1, d),
      scale, gamma.reshape(1, d), beta.reshape(1, d))
    return out


# speedup vs baseline: 1.8237x; 1.8222x over previous
import functools

import jax
import jax.numpy as jnp
from jax import lax
from jax.experimental import pallas as pl
from jax.experimental.pallas import tpu as pltpu
from jax.experimental.pallas import tpu_sc as plsc

NC = 2
NS = 16
NW = NC * NS
CHUNK = 128


def _spmm_partials(x, src2d, dst2d, zeros):
    d = x.shape[1]
    npad = zeros.shape[0]
    r = src2d.shape[0]
    rows_per_sub = npad // NS
    iters = (r + NW - 1) // NW
    mesh = plsc.VectorSubcoreMesh(core_axis_name="c", subcore_axis_name="s")

    @functools.partial(
        pl.kernel,
        out_type=jax.ShapeDtypeStruct((NC, npad, d), jnp.float32),
        mesh=mesh,
        scratch_types=[
            pltpu.VMEM((CHUNK,), jnp.int32),
            pltpu.VMEM((CHUNK,), jnp.int32),
            pltpu.VMEM((CHUNK, d), jnp.float32),
            pltpu.VMEM_SHARED((npad, d), jnp.float32),
            pltpu.SemaphoreType.DMA,
        ],
    )
    def spmm(x_hbm, src_hbm, dst_hbm, zero_hbm, out_hbm,
             sidx, didx, rows, yacc, sem):
        c = lax.axis_index("c")
        s = lax.axis_index("s")
        wid = s * NC + c
        pltpu.sync_copy(zero_hbm.at[pl.ds(s * rows_per_sub, rows_per_sub)],
                        yacc.at[pl.ds(s * rows_per_sub, rows_per_sub)])
        plsc.subcore_barrier()

        def body(it, carry):
            row = wid + it * NW

            @pl.when(row < r)
            def _():
                pltpu.sync_copy(src_hbm.at[row], sidx)
                pltpu.sync_copy(dst_hbm.at[row], didx)
                pltpu.async_copy(x_hbm.at[sidx], rows, sem).wait()
                pltpu.sync_copy(rows, yacc.at[didx], add=True)

            return carry

        lax.fori_loop(0, iters, body, 0)
        plsc.subcore_barrier()
        pltpu.sync_copy(yacc.at[pl.ds(s * rows_per_sub, rows_per_sub)],
                        out_hbm.at[c, pl.ds(s * rows_per_sub, rows_per_sub)])

    return spmm(x, src2d, dst2d, zeros)


def _mlp_body(y_ref, x_ref, w1_ref, b1_ref, w2_ref, b2_ref, scale_ref,
              g_ref, bt_ref, o_ref):
    n = x_ref.shape[0]
    h = y_ref[0, :n] + y_ref[1, :n] + scale_ref[...] * x_ref[...]
    h = jnp.dot(h, w1_ref[...], preferred_element_type=jnp.float32)
    h = jnp.maximum(h + b1_ref[...], 0.0)
    h = jnp.dot(h, w2_ref[...], preferred_element_type=jnp.float32)
    h = h + b2_ref[...]
    mean = jnp.mean(h, axis=0, keepdims=True)
    var = jnp.mean(jnp.square(h - mean), axis=0, keepdims=True)
    h = (h - mean) * lax.rsqrt(var + 1e-5) * g_ref[...] + bt_ref[...]
    o_ref[...] = jnp.maximum(h, 0.0)


def kernel(x, edge_index, W1, b1, W2, b2, eps, gamma, beta):
    n, d = x.shape
    e = edge_index.shape[1]
    assert e % CHUNK == 0
    npad = ((n + 8 * NS - 1) // (8 * NS)) * (8 * NS)
    dst2d = edge_index[0].reshape(e // CHUNK, CHUNK)
    src2d = edge_index[1].reshape(e // CHUNK, CHUNK)
    zeros = jnp.zeros((npad, d), jnp.float32)

    partials = _spmm_partials(x, src2d, dst2d, zeros)

    scale = (1.0 + eps).reshape(1, 1)
    out = pl.pallas_call(
        _mlp_body,
        out_shape=jax.ShapeDtypeStruct((n, d), jnp.float32),
    )(partials, x, W1.T, b1.reshape(1, d), W2.T, b2.reshape(1, d),
      scale, gamma.reshape(1, d), beta.reshape(1, d))
    return out


# speedup vs baseline: 2.0835x; 1.1424x over previous
import functools

import jax
import jax.numpy as jnp
from jax import lax
from jax.experimental import pallas as pl
from jax.experimental.pallas import tpu as pltpu
from jax.experimental.pallas import tpu_sc as plsc

NC = 2
NS = 16
NW = NC * NS
CHUNK = 128


def _spmm_partials(x, src2d, dst2d, zeros):
    d = x.shape[1]
    npad = zeros.shape[0]
    r = src2d.shape[0]
    rows_per_sub = npad // NS
    iters = (r + NW - 1) // NW
    mesh = plsc.VectorSubcoreMesh(core_axis_name="c", subcore_axis_name="s")

    @functools.partial(
        pl.kernel,
        out_type=jax.ShapeDtypeStruct((NC, npad, d), jnp.float32),
        mesh=mesh,
        scratch_types=[
            pltpu.VMEM((CHUNK,), jnp.int32),
            pltpu.VMEM((CHUNK,), jnp.int32),
            pltpu.VMEM((CHUNK, d), jnp.float32),
            pltpu.VMEM_SHARED((npad, d), jnp.float32),
            pltpu.SemaphoreType.DMA,
        ],
    )
    def spmm(x_hbm, src_hbm, dst_hbm, zero_hbm, out_hbm,
             sidx, didx, rows, yacc, sem):
        c = lax.axis_index("c")
        s = lax.axis_index("s")
        wid = s * NC + c
        pltpu.sync_copy(zero_hbm.at[pl.ds(s * rows_per_sub, rows_per_sub)],
                        yacc.at[pl.ds(s * rows_per_sub, rows_per_sub)])
        plsc.subcore_barrier()

        def body(it, carry):
            row = wid + it * NW

            @pl.when(row < r)
            def _():
                pltpu.sync_copy(src_hbm.at[row], sidx)
                gat = pltpu.async_copy(x_hbm.at[sidx], rows, sem)
                pltpu.sync_copy(dst_hbm.at[row], didx)
                gat.wait()
                pltpu.sync_copy(rows, yacc.at[didx], add=True)

            return carry

        lax.fori_loop(0, iters, body, 0)
        plsc.subcore_barrier()
        pltpu.sync_copy(yacc.at[pl.ds(s * rows_per_sub, rows_per_sub)],
                        out_hbm.at[c, pl.ds(s * rows_per_sub, rows_per_sub)])

    return spmm(x, src2d, dst2d, zeros)


def _mlp_body(y_ref, x_ref, w1_ref, b1_ref, w2_ref, b2_ref, scale_ref,
              g_ref, bt_ref, o_ref):
    n = x_ref.shape[0]
    h = y_ref[0, :n] + y_ref[1, :n] + scale_ref[...] * x_ref[...]
    h = jnp.dot(h, w1_ref[...], preferred_element_type=jnp.float32)
    h = jnp.maximum(h + b1_ref[...], 0.0)
    h = jnp.dot(h, w2_ref[...], preferred_element_type=jnp.float32)
    h = h + b2_ref[...]
    mean = jnp.mean(h, axis=0, keepdims=True)
    var = jnp.mean(jnp.square(h - mean), axis=0, keepdims=True)
    h = (h - mean) * lax.rsqrt(var + 1e-5) * g_ref[...] + bt_ref[...]
    o_ref[...] = jnp.maximum(h, 0.0)


def kernel(x, edge_index, W1, b1, W2, b2, eps, gamma, beta):
    n, d = x.shape
    e = edge_index.shape[1]
    assert e % CHUNK == 0
    npad = ((n + 8 * NS - 1) // (8 * NS)) * (8 * NS)
    dst2d = edge_index[0].reshape(e // CHUNK, CHUNK)
    src2d = edge_index[1].reshape(e // CHUNK, CHUNK)
    zeros = jnp.zeros((npad, d), jnp.float32)

    partials = _spmm_partials(x, src2d, dst2d, zeros)

    scale = (1.0 + eps).reshape(1, 1)
    out = pl.pallas_call(
        _mlp_body,
        out_shape=jax.ShapeDtypeStruct((n, d), jnp.float32),
    )(partials, x, W1.T, b1.reshape(1, d), W2.T, b2.reshape(1, d),
      scale, gamma.reshape(1, d), beta.reshape(1, d))
    return out


# speedup vs baseline: 2.4106x; 1.1570x over previous
import functools

import jax
import jax.numpy as jnp
from jax import lax
from jax.experimental import pallas as pl
from jax.experimental.pallas import tpu as pltpu
from jax.experimental.pallas import tpu_sc as plsc

NC = 2
NS = 16
NW = NC * NS
CHUNK = 128


def _spmm_partials(x, src2d, dst2d, zeros):
    d = x.shape[1]
    npad = zeros.shape[0]
    r = src2d.shape[0]
    rows_per_sub = npad // NS
    iters = (r + NW - 1) // NW
    mesh = plsc.VectorSubcoreMesh(core_axis_name="c", subcore_axis_name="s")

    @functools.partial(
        pl.kernel,
        out_type=jax.ShapeDtypeStruct((NC, npad, d), jnp.float32),
        mesh=mesh,
        scratch_types=[
            pltpu.VMEM((CHUNK,), jnp.int32),
            pltpu.VMEM((CHUNK,), jnp.int32),
            pltpu.VMEM((CHUNK, d), jnp.float32),
            pltpu.VMEM_SHARED((npad, d), jnp.float32),
            pltpu.SemaphoreType.DMA,
            pltpu.SemaphoreType.DMA,
        ],
    )
    def spmm(x_hbm, src_hbm, dst_hbm, zero_hbm, out_hbm,
             sidx, didx, rows, yacc, sem, isem):
        c = lax.axis_index("c")
        s = lax.axis_index("s")
        wid = s * NC + c
        pltpu.sync_copy(zero_hbm.at[pl.ds(s * rows_per_sub, rows_per_sub)],
                        yacc.at[pl.ds(s * rows_per_sub, rows_per_sub)])
        plsc.subcore_barrier()

        pltpu.async_copy(src_hbm.at[wid], sidx, isem)

        def body(it, carry):
            row = wid + it * NW

            @pl.when(row < r)
            def _():
                pltpu.make_async_copy(src_hbm.at[row], sidx, isem).wait()
                gat = pltpu.async_copy(x_hbm.at[sidx], rows, sem)
                pltpu.sync_copy(dst_hbm.at[row], didx)
                gat.wait()
                nrow = row + NW

                @pl.when(nrow < r)
                def _p():
                    pltpu.async_copy(src_hbm.at[nrow], sidx, isem)

                pltpu.sync_copy(rows, yacc.at[didx], add=True)

            return carry

        lax.fori_loop(0, iters, body, 0)
        plsc.subcore_barrier()
        pltpu.sync_copy(yacc.at[pl.ds(s * rows_per_sub, rows_per_sub)],
                        out_hbm.at[c, pl.ds(s * rows_per_sub, rows_per_sub)])

    return spmm(x, src2d, dst2d, zeros)


def _mlp_body(y_ref, x_ref, w1_ref, b1_ref, w2_ref, b2_ref, scale_ref,
              g_ref, bt_ref, o_ref):
    n = x_ref.shape[0]
    h = y_ref[0, :n] + y_ref[1, :n] + scale_ref[...] * x_ref[...]
    h = jnp.dot(h, w1_ref[...], preferred_element_type=jnp.float32)
    h = jnp.maximum(h + b1_ref[...], 0.0)
    h = jnp.dot(h, w2_ref[...], preferred_element_type=jnp.float32)
    h = h + b2_ref[...]
    mean = jnp.mean(h, axis=0, keepdims=True)
    var = jnp.mean(jnp.square(h - mean), axis=0, keepdims=True)
    h = (h - mean) * lax.rsqrt(var + 1e-5) * g_ref[...] + bt_ref[...]
    o_ref[...] = jnp.maximum(h, 0.0)


def kernel(x, edge_index, W1, b1, W2, b2, eps, gamma, beta):
    n, d = x.shape
    e = edge_index.shape[1]
    assert e % CHUNK == 0
    npad = ((n + 8 * NS - 1) // (8 * NS)) * (8 * NS)
    dst2d = edge_index[0].reshape(e // CHUNK, CHUNK)
    src2d = edge_index[1].reshape(e // CHUNK, CHUNK)
    zeros = jnp.zeros((npad, d), jnp.float32)

    partials = _spmm_partials(x, src2d, dst2d, zeros)

    scale = (1.0 + eps).reshape(1, 1)
    out = pl.pallas_call(
        _mlp_body,
        out_shape=jax.ShapeDtypeStruct((n, d), jnp.float32),
    )(partials, x, W1.T, b1.reshape(1, d), W2.T, b2.reshape(1, d),
      scale, gamma.reshape(1, d), beta.reshape(1, d))
    return out


# speedup vs baseline: 2.6834x; 1.1132x over previous
import functools

import jax
import jax.numpy as jnp
from jax import lax
from jax.experimental import pallas as pl
from jax.experimental.pallas import tpu as pltpu
from jax.experimental.pallas import tpu_sc as plsc

NC = 2
NS = 16
NW = NC * NS
CHUNK = 128


def _spmm_partials(x, src2d, dst2d, zeros):
    d = x.shape[1]
    npad = zeros.shape[0]
    r = src2d.shape[0]
    rows_per_sub = npad // NS
    iters = (r + NW - 1) // NW
    mesh = plsc.VectorSubcoreMesh(core_axis_name="c", subcore_axis_name="s")

    @functools.partial(
        pl.kernel,
        out_type=jax.ShapeDtypeStruct((NC, npad, d), jnp.float32),
        mesh=mesh,
        scratch_types=[
            pltpu.VMEM((2 * CHUNK,), jnp.int32),
            pltpu.VMEM((CHUNK,), jnp.int32),
            pltpu.VMEM((CHUNK,), jnp.int32),
            pltpu.VMEM((2 * CHUNK, d), jnp.float32),
            pltpu.VMEM_SHARED((npad, d), jnp.float32),
            pltpu.SemaphoreType.DMA,
            pltpu.SemaphoreType.DMA,
        ],
    )
    def spmm(x_hbm, src_hbm, dst_hbm, zero_hbm, out_hbm,
             sidx, didxa, didxb, rows, yacc, sem, isem):
        c = lax.axis_index("c")
        s = lax.axis_index("s")
        wid = s * NC + c
        pltpu.sync_copy(zero_hbm.at[pl.ds(s * rows_per_sub, rows_per_sub)],
                        yacc.at[pl.ds(s * rows_per_sub, rows_per_sub)])
        plsc.subcore_barrier()

        pltpu.async_copy(src_hbm.at[wid], sidx, isem)

        def body(it, carry):
            row = wid + it * NW

            @pl.when(row < r)
            def _():
                pltpu.make_async_copy(src_hbm.at[row], sidx, isem).wait()
                gat = pltpu.async_copy(x_hbm.at[sidx], rows, sem)
                pltpu.sync_copy(dst_hbm.at[2 * row], didxa)
                pltpu.sync_copy(dst_hbm.at[2 * row + 1], didxb)
                gat.wait()
                nrow = row + NW

                @pl.when(nrow < r)
                def _p():
                    pltpu.async_copy(src_hbm.at[nrow], sidx, isem)

                pltpu.sync_copy(rows.at[pl.ds(0, CHUNK)],
                                yacc.at[didxa], add=True)
                pltpu.sync_copy(rows.at[pl.ds(CHUNK, CHUNK)],
                                yacc.at[didxb], add=True)

            return carry

        lax.fori_loop(0, iters, body, 0)
        plsc.subcore_barrier()
        pltpu.sync_copy(yacc.at[pl.ds(s * rows_per_sub, rows_per_sub)],
                        out_hbm.at[c, pl.ds(s * rows_per_sub, rows_per_sub)])

    return spmm(x, src2d, dst2d, zeros)


def _mlp_body(y_ref, x_ref, w1_ref, b1_ref, w2_ref, b2_ref, scale_ref,
              g_ref, bt_ref, o_ref):
    n = x_ref.shape[0]
    h = y_ref[0, :n] + y_ref[1, :n] + scale_ref[...] * x_ref[...]
    h = jnp.dot(h, w1_ref[...], preferred_element_type=jnp.float32)
    h = jnp.maximum(h + b1_ref[...], 0.0)
    h = jnp.dot(h, w2_ref[...], preferred_element_type=jnp.float32)
    h = h + b2_ref[...]
    mean = jnp.mean(h, axis=0, keepdims=True)
    var = jnp.mean(jnp.square(h - mean), axis=0, keepdims=True)
    h = (h - mean) * lax.rsqrt(var + 1e-5) * g_ref[...] + bt_ref[...]
    o_ref[...] = jnp.maximum(h, 0.0)


def kernel(x, edge_index, W1, b1, W2, b2, eps, gamma, beta):
    n, d = x.shape
    e = edge_index.shape[1]
    assert e % (2 * CHUNK) == 0
    npad = ((n + 8 * NS - 1) // (8 * NS)) * (8 * NS)
    dst2d = edge_index[0].reshape(e // CHUNK, CHUNK)
    src2d = edge_index[1].reshape(e // (2 * CHUNK), 2 * CHUNK)
    zeros = jnp.zeros((npad, d), jnp.float32)

    partials = _spmm_partials(x, src2d, dst2d, zeros)

    scale = (1.0 + eps).reshape(1, 1)
    out = pl.pallas_call(
        _mlp_body,
        out_shape=jax.ShapeDtypeStruct((n, d), jnp.float32),
    )(partials, x, W1.T, b1.reshape(1, d), W2.T, b2.reshape(1, d),
      scale, gamma.reshape(1, d), beta.reshape(1, d))
    return out
